# Initial kernel scaffold; baseline (speedup 1.0000x reference)
import jax
import jax.numpy as jnp
from jax.experimental import pallas as pl


def kernel(feature, edge_index, W_fc, W_attn):
    raise NotImplementedError("write your pallas kernel here")


# speedup vs baseline: 10.3170x; 10.3170x over previous
import functools

import jax
import jax.numpy as jnp
from jax import lax
from jax.experimental import pallas as pl
from jax.experimental.pallas import tpu as pltpu
from jax.experimental.pallas import tpu_sc as plsc

N = 10000
E = 320000
D = 128
NC = 2
NS = 16
NW = NC * NS
L = 16
NP = 10240
RPT = NP // NS
EPW = E // NW
CH = 80
NCHUNK = EPW // CH

_mesh = plsc.VectorSubcoreMesh(core_axis_name="c", subcore_axis_name="s")


def _prep_body(f_ref, wfc_ref, wat_ref, z_ref, sl_ref, sr_ref):
    z = lax.dot_general(f_ref[...], wfc_ref[...], (((1,), (1,)), ((), ())),
                        preferred_element_type=jnp.float32)
    z_ref[...] = z
    wat = wat_ref[...]
    sl_ref[...] = lax.dot_general(z, wat[:, :D], (((1,), (1,)), ((), ())),
                                  preferred_element_type=jnp.float32)
    sr_ref[...] = lax.dot_general(z, wat[:, D:], (((1,), (1,)), ((), ())),
                                  preferred_element_type=jnp.float32)


_prep = pl.pallas_call(
    _prep_body,
    out_shape=[
        jax.ShapeDtypeStruct((N, D), jnp.float32),
        jax.ShapeDtypeStruct((N, 1), jnp.float32),
        jax.ShapeDtypeStruct((N, 1), jnp.float32),
    ],
)


@functools.partial(
    pl.kernel,
    out_type=[
        jax.ShapeDtypeStruct((E,), jnp.float32),
        jax.ShapeDtypeStruct((NC, NP), jnp.float32),
    ],
    mesh=_mesh,
    scratch_types=[
        pltpu.VMEM((CH,), jnp.int32),
        pltpu.VMEM((CH,), jnp.int32),
        pltpu.VMEM((CH,), jnp.float32),
        pltpu.VMEM((CH,), jnp.float32),
        pltpu.VMEM((CH,), jnp.float32),
        pltpu.VMEM_SHARED((NP,), jnp.float32),
        pltpu.SemaphoreType.DMA,
        pltpu.SemaphoreType.DMA,
    ],
)
def _sc_edge(sl_hbm, sr_hbm, src_hbm, dst_hbm, zvec_hbm,
             eexp_hbm, den_hbm,
             sidx, didx, v1, v2, pb, dacc, sem1, sem2):
    cid = lax.axis_index("c")
    sid = lax.axis_index("s")
    row0 = sid * RPT
    pltpu.sync_copy(zvec_hbm.at[pl.ds(row0, RPT)], dacc.at[pl.ds(row0, RPT)])
    plsc.subcore_barrier()

    wbase = (sid * NC + cid) * EPW

    @pl.loop(0, NCHUNK)
    def _chunk(j):
        base = wbase + j * CH
        pltpu.sync_copy(src_hbm.at[pl.ds(base, CH)], sidx)
        pltpu.sync_copy(dst_hbm.at[pl.ds(base, CH)], didx)
        ca = pltpu.async_copy(sl_hbm.at[sidx], v1, sem1)
        cb = pltpu.async_copy(sr_hbm.at[didx], v2, sem2)
        ca.wait()
        cb.wait()
        for k in range(CH // L):
            s = pl.ds(k * L, L)
            a = v1[s] + v2[s]
            e = jnp.where(a >= 0.0, a, a * jnp.float32(0.01))
            pb[s] = jnp.exp(e)
        pltpu.sync_copy(pb, eexp_hbm.at[pl.ds(base, CH)])
        pltpu.sync_copy(pb, dacc.at[sidx], add=True)

    plsc.subcore_barrier()

    @pl.when(sid == 0)
    def _dump():
        pltpu.sync_copy(dacc, den_hbm.at[cid])


@functools.partial(
    pl.kernel,
    out_type=jax.ShapeDtypeStruct((NC, NP, D), jnp.float32),
    mesh=_mesh,
    scratch_types=[
        pltpu.VMEM((CH,), jnp.int32),
        pltpu.VMEM((CH,), jnp.int32),
        pltpu.VMEM((CH,), jnp.float32),
        pltpu.VMEM((CH,), jnp.float32),
        pltpu.VMEM((CH,), jnp.float32),
        pltpu.VMEM((CH,), jnp.float32),
        pltpu.VMEM((CH, D), jnp.float32),
        pltpu.VMEM_SHARED((NP, D), jnp.float32),
        pltpu.SemaphoreType.DMA,
        pltpu.SemaphoreType.DMA,
        pltpu.SemaphoreType.DMA,
    ],
)
def _sc_agg(z_hbm, eexp_hbm, d0_hbm, d1_hbm, src_hbm, dst_hbm, zmat_hbm,
            hp_hbm,
            sidx, didx, pb, g0, g1, al, zr, hacc, semz, sem0, sem1):
    cid = lax.axis_index("c")
    sid = lax.axis_index("s")
    row0 = sid * RPT
    pltpu.sync_copy(zmat_hbm.at[pl.ds(row0, RPT)], hacc.at[pl.ds(row0, RPT)])
    plsc.subcore_barrier()

    wbase = (sid * NC + cid) * EPW

    @pl.loop(0, NCHUNK)
    def _chunk(j):
        base = wbase + j * CH
        pltpu.sync_copy(src_hbm.at[pl.ds(base, CH)], sidx)
        pltpu.sync_copy(dst_hbm.at[pl.ds(base, CH)], didx)
        pltpu.sync_copy(eexp_hbm.at[pl.ds(base, CH)], pb)
        cz = pltpu.async_copy(z_hbm.at[sidx], zr, semz)
        c0 = pltpu.async_copy(d0_hbm.at[sidx], g0, sem0)
        c1 = pltpu.async_copy(d1_hbm.at[sidx], g1, sem1)
        c0.wait()
        c1.wait()
        for k in range(CH // L):
            s = pl.ds(k * L, L)
            al[s] = pb[s] / (g0[s] + g1[s])
        cz.wait()

        @pl.loop(0, CH // L)
        def _grp(g):
            a16 = al[pl.ds(g * L, L)]
            for i in range(L):
                a = a16[i]
                r = g * L + i
                for k in range(D // L):
                    s = pl.ds(k * L, L)
                    zr[r, s] = zr[r, s] * a

        pltpu.sync_copy(zr, hacc.at[didx], add=True)

    plsc.subcore_barrier()
    pltpu.sync_copy(hacc.at[pl.ds(row0, RPT)],
                    hp_hbm.at[cid, pl.ds(row0, RPT)])


def _combine_body(a_ref, b_ref, o_ref):
    o_ref[...] = a_ref[...] + b_ref[...]


_combine = pl.pallas_call(
    _combine_body,
    grid=(5,),
    in_specs=[
        pl.BlockSpec((2000, D), lambda i: (i, 0)),
        pl.BlockSpec((2000, D), lambda i: (i, 0)),
    ],
    out_specs=pl.BlockSpec((2000, D), lambda i: (i, 0)),
    out_shape=jax.ShapeDtypeStruct((N, D), jnp.float32),
)


def kernel(feature, edge_index, W_fc, W_attn):
    src = edge_index[0].astype(jnp.int32)
    dst = edge_index[1].astype(jnp.int32)
    z, sl, sr = _prep(feature, W_fc, W_attn)
    sl = sl.reshape(N)
    sr = sr.reshape(N)
    zvec = jnp.zeros((NP,), jnp.float32)
    zmat = jnp.zeros((NP, D), jnp.float32)
    eexp, den = _sc_edge(sl, sr, src, dst, zvec)
    hp = _sc_agg(z, eexp, den[0], den[1], src, dst, zmat)
    return _combine(hp[0, :N], hp[1, :N])


# speedup vs baseline: 27.6216x; 2.6773x over previous
import functools

import jax
import jax.numpy as jnp
from jax import lax
from jax.experimental import pallas as pl
from jax.experimental.pallas import tpu as pltpu
from jax.experimental.pallas import tpu_sc as plsc

N = 10000
E = 320000
D = 128
NC = 2
NS = 16
NW = NC * NS
L = 16
NP = 10240
RPT = NP // NS
CH = 80
NR = E // NW // CH
SB = 25
NSC = NR // SB

_mesh = plsc.VectorSubcoreMesh(core_axis_name="c", subcore_axis_name="s")


def _prep_body(f_ref, wfc_ref, wat_ref, z_ref, sl_ref, sr_ref):
    z = lax.dot_general(f_ref[...], wfc_ref[...], (((1,), (1,)), ((), ())),
                        preferred_element_type=jnp.float32)
    z_ref[...] = z
    wat = wat_ref[...]
    sl_ref[...] = lax.dot_general(z, wat[:, :D], (((1,), (1,)), ((), ())),
                                  preferred_element_type=jnp.float32)
    sr_ref[...] = lax.dot_general(z, wat[:, D:], (((1,), (1,)), ((), ())),
                                  preferred_element_type=jnp.float32)


_prep = pl.pallas_call(
    _prep_body,
    out_shape=[
        jax.ShapeDtypeStruct((N, D), jnp.float32),
        jax.ShapeDtypeStruct((N, 1), jnp.float32),
        jax.ShapeDtypeStruct((N, 1), jnp.float32),
    ],
)


@functools.partial(
    pl.kernel,
    out_type=[
        jax.ShapeDtypeStruct((E,), jnp.float32),
        jax.ShapeDtypeStruct((NC, NP), jnp.float32),
    ],
    mesh=_mesh,
    scratch_types=[
        pltpu.VMEM((NR, CH), jnp.int32),
        pltpu.VMEM((NR, CH), jnp.int32),
        pltpu.VMEM((NR, CH), jnp.float32),
        pltpu.VMEM((NR, CH), jnp.float32),
        pltpu.VMEM((NR, CH), jnp.float32),
        pltpu.VMEM_SHARED((NP,), jnp.float32),
        pltpu.SemaphoreType.DMA,
        pltpu.SemaphoreType.DMA,
        pltpu.SemaphoreType.DMA,
    ],
)
def _sc_edge(sl_hbm, sr_hbm, src_hbm, dst_hbm, zvec_hbm,
             eexp_hbm, den_hbm,
             sblk, dblk, vl, vr, pblk, dacc, sem1, sem2, semsc):
    cid = lax.axis_index("c")
    sid = lax.axis_index("s")
    row0 = sid * RPT
    wid = sid * NC + cid
    wbase = wid * NR * CH

    @pl.loop(0, NR)
    def _ld(j):
        pltpu.async_copy(src_hbm.at[pl.ds(wbase + j * CH, CH)], sblk.at[j],
                         sem1)
        pltpu.async_copy(dst_hbm.at[pl.ds(wbase + j * CH, CH)], dblk.at[j],
                         sem2)

    pltpu.sync_copy(zvec_hbm.at[pl.ds(row0, RPT)], dacc.at[pl.ds(row0, RPT)])
    plsc.subcore_barrier()

    @pl.loop(0, NR)
    def _ldw(j):
        pltpu.make_async_copy(src_hbm.at[pl.ds(wbase + j * CH, CH)],
                              sblk.at[j], sem1).wait()
        pltpu.make_async_copy(dst_hbm.at[pl.ds(wbase + j * CH, CH)],
                              dblk.at[j], sem2).wait()

    @pl.loop(0, NR)
    def _fire(j):
        pltpu.async_copy(sl_hbm.at[sblk.at[j]], vl.at[j], sem1)
        pltpu.async_copy(sr_hbm.at[dblk.at[j]], vr.at[j], sem2)

    @pl.loop(0, NR)
    def _r(j):
        pltpu.make_async_copy(sl_hbm.at[sblk.at[j]], vl.at[j], sem1).wait()
        pltpu.make_async_copy(sr_hbm.at[dblk.at[j]], vr.at[j], sem2).wait()
        for k in range(CH // L):
            s = pl.ds(k * L, L)
            a = vl[j, s] + vr[j, s]
            e = jnp.where(a >= 0.0, a, a * jnp.float32(0.01))
            pblk[j, s] = jnp.exp(e)
        pltpu.async_copy(pblk.at[j], dacc.at[sblk.at[j]], semsc, add=True)
        pltpu.async_copy(pblk.at[j], eexp_hbm.at[pl.ds(wbase + j * CH, CH)],
                         sem2)

    @pl.loop(0, NR)
    def _dr(j):
        pltpu.make_async_copy(pblk.at[j], dacc.at[sblk.at[j]], semsc).wait()
        pltpu.make_async_copy(pblk.at[j],
                              eexp_hbm.at[pl.ds(wbase + j * CH, CH)],
                              sem2).wait()

    plsc.subcore_barrier()

    @pl.when(sid == 0)
    def _dump():
        pltpu.sync_copy(dacc, den_hbm.at[cid])


def _wscale_body(z_ref, d0_ref, d1_ref, w_ref):
    w_ref[...] = z_ref[...] * (1.0 / (d0_ref[...] + d1_ref[...]))


_wscale = pl.pallas_call(
    _wscale_body,
    grid=(5,),
    in_specs=[
        pl.BlockSpec((2000, D), lambda i: (i, 0)),
        pl.BlockSpec((2000, 1), lambda i: (i, 0)),
        pl.BlockSpec((2000, 1), lambda i: (i, 0)),
    ],
    out_specs=pl.BlockSpec((2000, D), lambda i: (i, 0)),
    out_shape=jax.ShapeDtypeStruct((N, D), jnp.float32),
)


@functools.partial(
    pl.kernel,
    out_type=jax.ShapeDtypeStruct((NC, NP, D), jnp.float32),
    mesh=_mesh,
    scratch_types=[
        pltpu.VMEM((SB * CH,), jnp.int32),
        pltpu.VMEM((SB, CH), jnp.int32),
        pltpu.VMEM((SB * CH,), jnp.float32),
        pltpu.VMEM((3, CH, D), jnp.float32),
        pltpu.VMEM_SHARED((NP, D), jnp.float32),
        pltpu.SemaphoreType.DMA,
        pltpu.SemaphoreType.DMA,
        pltpu.SemaphoreType.DMA,
    ],
)
def _sc_agg(w_hbm, eexp_hbm, src_hbm, dst_hbm, zmat_hbm,
            hp_hbm,
            sblk, dblk, pblk, zr, hacc, semz, semsc, sem1):
    cid = lax.axis_index("c")
    sid = lax.axis_index("s")
    row0 = sid * RPT
    wid = sid * NC + cid
    wbase = wid * NR * CH

    pltpu.sync_copy(zmat_hbm.at[pl.ds(row0, RPT)], hacc.at[pl.ds(row0, RPT)])
    plsc.subcore_barrier()

    @pl.loop(0, NSC)
    def _super(sc):
        base = wbase + sc * SB * CH
        cs = pltpu.async_copy(src_hbm.at[pl.ds(base, SB * CH)], sblk, sem1)
        cp = pltpu.async_copy(eexp_hbm.at[pl.ds(base, SB * CH)], pblk, semz)

        @pl.loop(0, SB)
        def _ldd(jj):
            pltpu.async_copy(dst_hbm.at[pl.ds(base + jj * CH, CH)],
                             dblk.at[jj], semsc)

        cs.wait()
        cp.wait()

        @pl.loop(0, SB)
        def _ldw(jj):
            pltpu.make_async_copy(dst_hbm.at[pl.ds(base + jj * CH, CH)],
                                  dblk.at[jj], semsc).wait()

        pltpu.async_copy(w_hbm.at[sblk.at[pl.ds(0, CH)]], zr.at[0], semz)

        @pl.loop(0, SB)
        def _row(jj):
            b = lax.rem(jj, 3)

            @pl.when(jj >= 2)
            def _drain_scatter():
                bd = lax.rem(jj + 1, 3)
                pltpu.make_async_copy(zr.at[bd], hacc.at[dblk.at[jj - 2]],
                                      semsc).wait()

            @pl.when(jj + 1 < SB)
            def _issue_gather():
                bn = lax.rem(jj + 1, 3)
                pltpu.async_copy(
                    w_hbm.at[sblk.at[pl.ds((jj + 1) * CH, CH)]],
                    zr.at[bn], semz)

            pltpu.make_async_copy(w_hbm.at[sblk.at[pl.ds(jj * CH, CH)]],
                                  zr.at[b], semz).wait()

            for g in range(CH // L):
                a16 = pblk[pl.ds(jj * CH + g * L, L)]
                for i in range(L):
                    r = g * L + i
                    for k in range(D // L):
                        s = pl.ds(k * L, L)
                        zr[b, r, s] = zr[b, r, s] * a16[i]

            pltpu.async_copy(zr.at[b], hacc.at[dblk.at[jj]], semsc, add=True)

        pltpu.make_async_copy(zr.at[(SB - 2) % 3], hacc.at[dblk.at[SB - 2]],
                              semsc).wait()
        pltpu.make_async_copy(zr.at[(SB - 1) % 3], hacc.at[dblk.at[SB - 1]],
                              semsc).wait()

    plsc.subcore_barrier()
    pltpu.sync_copy(hacc.at[pl.ds(row0, RPT)],
                    hp_hbm.at[cid, pl.ds(row0, RPT)])


def _combine_body(a_ref, b_ref, o_ref):
    o_ref[...] = a_ref[...] + b_ref[...]


_combine = pl.pallas_call(
    _combine_body,
    grid=(5,),
    in_specs=[
        pl.BlockSpec((2000, D), lambda i: (i, 0)),
        pl.BlockSpec((2000, D), lambda i: (i, 0)),
    ],
    out_specs=pl.BlockSpec((2000, D), lambda i: (i, 0)),
    out_shape=jax.ShapeDtypeStruct((N, D), jnp.float32),
)


def kernel(feature, edge_index, W_fc, W_attn):
    src = edge_index[0].astype(jnp.int32)
    dst = edge_index[1].astype(jnp.int32)
    z, sl, sr = _prep(feature, W_fc, W_attn)
    sl = sl.reshape(N)
    sr = sr.reshape(N)
    zvec = jnp.zeros((NP,), jnp.float32)
    zmat = jnp.zeros((NP, D), jnp.float32)
    eexp, den = _sc_edge(sl, sr, src, dst, zvec)
    d0 = den[0, :N].reshape(N, 1)
    d1 = den[1, :N].reshape(N, 1)
    w = _wscale(z, d0, d1)
    hp = _sc_agg(w, eexp, src, dst, zmat)
    return _combine(hp[0, :N], hp[1, :N])
